# baseline (speedup 1.0000x reference)
import functools

import jax
import jax.numpy as jnp
from jax.experimental import pallas as pl
from jax.experimental.pallas import tpu as pltpu

NUM_HEADS = 16
HEAD_DIM = 128
HD = NUM_HEADS * HEAD_DIM
SCALE = 0.08838834764831845
LOG2E = 1.4426950408889634
PAGE = 16

BQ = 256
BK = 256
GBK = 256
NEG = -1e30


def _fill_kernel(q_ref, k_ref, v_ref, o_ref):
    qb = pl.program_id(2)
    qv = q_ref[...] * (SCALE * LOG2E)

    def step_unmasked(j, carry):
        acc, m, l = carry
        kj = k_ref[pl.ds(j * BK, BK), :]
        s = jax.lax.dot_general(qv, kj, (((1,), (1,)), ((), ())),
                                preferred_element_type=jnp.float32)
        m_new = jnp.maximum(m, jnp.max(s, axis=1, keepdims=True))
        alpha = jnp.exp2(m - m_new)
        p = jnp.exp2(s - m_new)
        l_new = l * alpha + jnp.sum(p, axis=1, keepdims=True)
        vj = v_ref[pl.ds(j * BK, BK), :]
        acc_new = acc * alpha + jax.lax.dot_general(
            p, vj, (((1,), (0,)), ((), ())), preferred_element_type=jnp.float32)
        return acc_new, m_new, l_new

    acc0 = jnp.zeros((BQ, HEAD_DIM), jnp.float32)
    m0 = jnp.full((BQ, 1), NEG, jnp.float32)
    l0 = jnp.zeros((BQ, 1), jnp.float32)
    acc, m, l = jax.lax.fori_loop(0, qb, step_unmasked, (acc0, m0, l0))

    kj = k_ref[pl.ds(qb * BK, BK), :]
    s = jax.lax.dot_general(qv, kj, (((1,), (1,)), ((), ())),
                            preferred_element_type=jnp.float32)
    row = jax.lax.broadcasted_iota(jnp.int32, (BQ, BK), 0)
    col = jax.lax.broadcasted_iota(jnp.int32, (BQ, BK), 1)
    s = jnp.where(col <= row, s, NEG)
    m_new = jnp.maximum(m, jnp.max(s, axis=1, keepdims=True))
    alpha = jnp.exp2(m - m_new)
    p = jnp.exp2(s - m_new)
    l = l * alpha + jnp.sum(p, axis=1, keepdims=True)
    vj = v_ref[pl.ds(qb * BK, BK), :]
    acc = acc * alpha + jax.lax.dot_general(
        p, vj, (((1,), (0,)), ((), ())), preferred_element_type=jnp.float32)
    o_ref[...] = acc / l


def _fill_attention(q2, k2, v2, seq_len, num_seqs, total):
    qblocks = seq_len // BQ
    grid = (num_seqs, NUM_HEADS, qblocks)
    return pl.pallas_call(
        _fill_kernel,
        grid=grid,
        in_specs=[
            pl.BlockSpec((BQ, HEAD_DIM), lambda s, h, qb: (s * qblocks + qb, h)),
            pl.BlockSpec((seq_len, HEAD_DIM), lambda s, h, qb: (s, h)),
            pl.BlockSpec((seq_len, HEAD_DIM), lambda s, h, qb: (s, h)),
        ],
        out_specs=pl.BlockSpec((BQ, HEAD_DIM), lambda s, h, qb: (s * qblocks + qb, h)),
        out_shape=jax.ShapeDtypeStruct((total, HD), jnp.float32),
    )(q2, k2, v2)


def _gen_kernel(bt_ref, cl_ref, io_ref, qbd_ref, kcur_ref, vcur_ref,
                kc_ref, vc_ref, o_ref, acc_ref, m_ref, l_ref, *, n_chunks):
    del io_ref
    g = pl.program_id(0)
    c = pl.program_id(1)
    ctx = cl_ref[g]
    qbd = qbd_ref[0]

    @pl.when(c == 0)
    def _init():
        logit_cur = jax.lax.dot_general(
            qbd, kcur_ref[0], (((0,), (1,)), ((), ())),
            preferred_element_type=jnp.float32)
        m_ref[...] = jnp.broadcast_to(logit_cur, (NUM_HEADS, HEAD_DIM))
        l_ref[...] = jnp.ones((NUM_HEADS, HEAD_DIM), jnp.float32)
        acc_ref[...] = jnp.broadcast_to(vcur_ref[0], (NUM_HEADS, HD))

    kblk = kc_ref[0]
    s_hs = jax.lax.dot_general(qbd, kblk, (((0,), (1,)), ((), ())),
                               preferred_element_type=jnp.float32)
    pos = c * GBK + jax.lax.broadcasted_iota(jnp.int32, (NUM_HEADS, GBK), 1)
    s_hs = jnp.where(pos < ctx - 1, s_hs, NEG)

    m_prev = m_ref[:, :1]
    l_prev = l_ref[:, :1]
    m_new = jnp.maximum(m_prev, jnp.max(s_hs, axis=1, keepdims=True))
    alpha = jnp.exp2(m_prev - m_new)
    p = jnp.exp2(s_hs - m_new)
    l_new = l_prev * alpha + jnp.sum(p, axis=1, keepdims=True)
    vblk = vc_ref[0]
    pv = jax.lax.dot_general(p, vblk, (((1,), (0,)), ((), ())),
                             preferred_element_type=jnp.float32)
    acc_new = acc_ref[...] * alpha + pv
    acc_ref[...] = acc_new
    m_ref[...] = jnp.broadcast_to(m_new, (NUM_HEADS, HEAD_DIM))
    l_ref[...] = jnp.broadcast_to(l_new, (NUM_HEADS, HEAD_DIM))

    @pl.when(c == n_chunks - 1)
    def _done():
        hsel = jax.lax.broadcasted_iota(jnp.int32, (NUM_HEADS, HD), 1) // HEAD_DIM
        hrow = jax.lax.broadcasted_iota(jnp.int32, (NUM_HEADS, HD), 0)
        picked = jnp.where(hsel == hrow, acc_new / l_new, 0.0)
        o_ref[0] = jnp.sum(picked, axis=0, keepdims=True)


def _gen_attention(fill_out, qg, kcur2, vcur2, kc, vc, block_tables,
                   context_lens, num_fill):
    n_gen, bpseq = block_tables.shape
    ctx_cap = bpseq * PAGE
    n_chunks = ctx_cap // GBK
    per = GBK // PAGE
    ngroups = kc.shape[0] // GBK
    kcg = kc.reshape(ngroups, GBK, HD)
    vcg = vc.reshape(ngroups, GBK, HD)

    ridx = jnp.arange(HD, dtype=jnp.int32)
    hmask = (ridx[:, None] // HEAD_DIM) == jnp.arange(NUM_HEADS, dtype=jnp.int32)[None, :]
    qbd = jnp.where(hmask[None], (qg * (SCALE * LOG2E))[:, :, None], 0.0)

    def cache_map(g, c, bt, cl):
        return (bt[g, c * per] // per, 0, 0)

    grid_spec = pltpu.PrefetchScalarGridSpec(
        num_scalar_prefetch=2,
        grid=(n_gen, n_chunks),
        in_specs=[
            pl.BlockSpec(memory_space=pl.ANY),
            pl.BlockSpec((1, HD, NUM_HEADS), lambda g, c, bt, cl: (g, 0, 0)),
            pl.BlockSpec((1, 1, HD), lambda g, c, bt, cl: (g, 0, 0)),
            pl.BlockSpec((1, 1, HD), lambda g, c, bt, cl: (g, 0, 0)),
            pl.BlockSpec((1, GBK, HD), cache_map),
            pl.BlockSpec((1, GBK, HD), cache_map),
        ],
        out_specs=pl.BlockSpec((1, 1, HD), lambda g, c, bt, cl: (num_fill + g, 0, 0)),
        scratch_shapes=[
            pltpu.VMEM((NUM_HEADS, HD), jnp.float32),
            pltpu.VMEM((NUM_HEADS, HEAD_DIM), jnp.float32),
            pltpu.VMEM((NUM_HEADS, HEAD_DIM), jnp.float32),
        ],
    )
    total = num_fill + n_gen
    out = pl.pallas_call(
        functools.partial(_gen_kernel, n_chunks=n_chunks),
        grid_spec=grid_spec,
        out_shape=jax.ShapeDtypeStruct((total, 1, HD), jnp.float32),
        input_output_aliases={2: 0},
    )(block_tables, context_lens, fill_out.reshape(total, 1, HD), qbd,
      kcur2.reshape(n_gen, 1, HD), vcur2.reshape(n_gen, 1, HD), kcg, vcg)
    return out.reshape(total, HD)


def kernel(q, k, v, k_cache, v_cache, slot_mapping, cu_seqlens_q,
           context_lens, block_tables):
    total = q.shape[0]
    n_gen = block_tables.shape[0]
    num_fill = total - n_gen
    num_seqs = cu_seqlens_q.shape[0] - 1
    seq_len = num_fill // num_seqs

    q2 = q.reshape(total, HD)
    k2 = k.reshape(total, HD)
    v2 = v.reshape(total, HD)

    fill_out = _fill_attention(q2, k2, v2, seq_len, num_seqs, total)
    out = _gen_attention(fill_out, q2[num_fill:], k2[num_fill:], v2[num_fill:],
                         k_cache.reshape(-1, HD), v_cache.reshape(-1, HD),
                         block_tables, context_lens, num_fill)
    return out


# speedup vs baseline: 1.0172x; 1.0172x over previous
import functools

import jax
import jax.numpy as jnp
from jax.experimental import pallas as pl
from jax.experimental.pallas import tpu as pltpu

NUM_HEADS = 16
HEAD_DIM = 128
HD = NUM_HEADS * HEAD_DIM
SCALE = 0.08838834764831845
LOG2E = 1.4426950408889634
PAGE = 16

BQ = 256
BK = 256
GBK = 512
NEG = -1e30


def _fill_kernel(q_ref, k_ref, v_ref, o_ref, kbf_ref, vbf_ref):
    qb = pl.program_id(2)

    @pl.when(qb == 0)
    def _cast():
        kbf_ref[...] = k_ref[...].astype(jnp.bfloat16)
        vbf_ref[...] = v_ref[...].astype(jnp.bfloat16)

    qv = (q_ref[...] * (SCALE * LOG2E)).astype(jnp.bfloat16)

    def step_unmasked(j, carry):
        acc, m, l = carry
        kj = kbf_ref[pl.ds(j * BK, BK), :]
        s = jax.lax.dot_general(qv, kj, (((1,), (1,)), ((), ())),
                                preferred_element_type=jnp.float32)
        m_new = jnp.maximum(m, jnp.max(s, axis=1, keepdims=True))
        alpha = jnp.exp2(m - m_new)
        p = jnp.exp2(s - m_new)
        l_new = l * alpha + jnp.sum(p, axis=1, keepdims=True)
        vj = vbf_ref[pl.ds(j * BK, BK), :]
        acc_new = acc * alpha + jax.lax.dot_general(
            p.astype(jnp.bfloat16), vj, (((1,), (0,)), ((), ())),
            preferred_element_type=jnp.float32)
        return acc_new, m_new, l_new

    acc0 = jnp.zeros((BQ, HEAD_DIM), jnp.float32)
    m0 = jnp.full((BQ, 1), NEG, jnp.float32)
    l0 = jnp.zeros((BQ, 1), jnp.float32)
    acc, m, l = jax.lax.fori_loop(0, qb, step_unmasked, (acc0, m0, l0))

    kj = kbf_ref[pl.ds(qb * BK, BK), :]
    s = jax.lax.dot_general(qv, kj, (((1,), (1,)), ((), ())),
                            preferred_element_type=jnp.float32)
    row = jax.lax.broadcasted_iota(jnp.int32, (BQ, BK), 0)
    col = jax.lax.broadcasted_iota(jnp.int32, (BQ, BK), 1)
    s = jnp.where(col <= row, s, NEG)
    m_new = jnp.maximum(m, jnp.max(s, axis=1, keepdims=True))
    alpha = jnp.exp2(m - m_new)
    p = jnp.exp2(s - m_new)
    l = l * alpha + jnp.sum(p, axis=1, keepdims=True)
    vj = vbf_ref[pl.ds(qb * BK, BK), :]
    acc = acc * alpha + jax.lax.dot_general(
        p.astype(jnp.bfloat16), vj, (((1,), (0,)), ((), ())),
        preferred_element_type=jnp.float32)
    o_ref[...] = acc / l


def _fill_attention(q2, k2, v2, seq_len, num_seqs, total):
    qblocks = seq_len // BQ
    grid = (num_seqs, NUM_HEADS, qblocks)
    return pl.pallas_call(
        _fill_kernel,
        grid=grid,
        in_specs=[
            pl.BlockSpec((BQ, HEAD_DIM), lambda s, h, qb: (s * qblocks + qb, h)),
            pl.BlockSpec((seq_len, HEAD_DIM), lambda s, h, qb: (s, h)),
            pl.BlockSpec((seq_len, HEAD_DIM), lambda s, h, qb: (s, h)),
        ],
        out_specs=pl.BlockSpec((BQ, HEAD_DIM), lambda s, h, qb: (s * qblocks + qb, h)),
        out_shape=jax.ShapeDtypeStruct((total, HD), jnp.float32),
        scratch_shapes=[
            pltpu.VMEM((seq_len, HEAD_DIM), jnp.bfloat16),
            pltpu.VMEM((seq_len, HEAD_DIM), jnp.bfloat16),
        ],
        compiler_params=pltpu.CompilerParams(
            dimension_semantics=("parallel", "arbitrary", "arbitrary")),
    )(q2, k2, v2)


def _gen_kernel(bt_ref, cl_ref, io_ref, qbd_ref, kcur_ref, vcur_ref,
                kc_ref, vc_ref, o_ref, acc_ref, m_ref, l_ref, *, n_chunks):
    del io_ref
    g = pl.program_id(0)
    c = pl.program_id(1)
    ctx = cl_ref[g]
    qbd = qbd_ref[0]

    @pl.when(c == 0)
    def _init():
        logit_cur = jax.lax.dot_general(
            qbd, kcur_ref[0], (((0,), (1,)), ((), ())),
            preferred_element_type=jnp.float32)
        m_ref[...] = jnp.broadcast_to(logit_cur, (NUM_HEADS, HEAD_DIM))
        l_ref[...] = jnp.ones((NUM_HEADS, HEAD_DIM), jnp.float32)
        acc_ref[...] = jnp.broadcast_to(vcur_ref[0], (NUM_HEADS, HD))

    kblk = kc_ref[0]
    s_hs = jax.lax.dot_general(qbd, kblk, (((0,), (1,)), ((), ())),
                               preferred_element_type=jnp.float32)
    pos = c * GBK + jax.lax.broadcasted_iota(jnp.int32, (NUM_HEADS, GBK), 1)
    s_hs = jnp.where(pos < ctx - 1, s_hs, NEG)

    m_prev = m_ref[:, :1]
    l_prev = l_ref[:, :1]
    m_new = jnp.maximum(m_prev, jnp.max(s_hs, axis=1, keepdims=True))
    alpha = jnp.exp2(m_prev - m_new)
    p = jnp.exp2(s_hs - m_new)
    l_new = l_prev * alpha + jnp.sum(p, axis=1, keepdims=True)
    vblk = vc_ref[0]
    pv = jax.lax.dot_general(p, vblk, (((1,), (0,)), ((), ())),
                             preferred_element_type=jnp.float32)
    acc_new = acc_ref[...] * alpha + pv
    acc_ref[...] = acc_new
    m_ref[...] = jnp.broadcast_to(m_new, (NUM_HEADS, HEAD_DIM))
    l_ref[...] = jnp.broadcast_to(l_new, (NUM_HEADS, HEAD_DIM))

    @pl.when(c == n_chunks - 1)
    def _done():
        hsel = jax.lax.broadcasted_iota(jnp.int32, (NUM_HEADS, HD), 1) // HEAD_DIM
        hrow = jax.lax.broadcasted_iota(jnp.int32, (NUM_HEADS, HD), 0)
        picked = jnp.where(hsel == hrow, acc_new / l_new, 0.0)
        o_ref[0] = jnp.sum(picked, axis=0, keepdims=True)


def _gen_attention(fill_out, qg, kcur2, vcur2, kc, vc, block_tables,
                   context_lens, num_fill):
    n_gen, bpseq = block_tables.shape
    ctx_cap = bpseq * PAGE
    n_chunks = ctx_cap // GBK
    per = GBK // PAGE
    ngroups = kc.shape[0] // GBK
    kcg = kc.reshape(ngroups, GBK, HD)
    vcg = vc.reshape(ngroups, GBK, HD)

    ridx = jnp.arange(HD, dtype=jnp.int32)
    hmask = (ridx[:, None] // HEAD_DIM) == jnp.arange(NUM_HEADS, dtype=jnp.int32)[None, :]
    qbd = jnp.where(hmask[None], (qg * (SCALE * LOG2E))[:, :, None], 0.0)

    def cache_map(g, c, bt, cl):
        return (bt[g, c * per] // per, 0, 0)

    grid_spec = pltpu.PrefetchScalarGridSpec(
        num_scalar_prefetch=2,
        grid=(n_gen, n_chunks),
        in_specs=[
            pl.BlockSpec(memory_space=pl.ANY),
            pl.BlockSpec((1, HD, NUM_HEADS), lambda g, c, bt, cl: (g, 0, 0)),
            pl.BlockSpec((1, 1, HD), lambda g, c, bt, cl: (g, 0, 0)),
            pl.BlockSpec((1, 1, HD), lambda g, c, bt, cl: (g, 0, 0)),
            pl.BlockSpec((1, GBK, HD), cache_map),
            pl.BlockSpec((1, GBK, HD), cache_map),
        ],
        out_specs=pl.BlockSpec((1, 1, HD), lambda g, c, bt, cl: (num_fill + g, 0, 0)),
        scratch_shapes=[
            pltpu.VMEM((NUM_HEADS, HD), jnp.float32),
            pltpu.VMEM((NUM_HEADS, HEAD_DIM), jnp.float32),
            pltpu.VMEM((NUM_HEADS, HEAD_DIM), jnp.float32),
        ],
    )
    total = num_fill + n_gen
    out = pl.pallas_call(
        functools.partial(_gen_kernel, n_chunks=n_chunks),
        grid_spec=grid_spec,
        out_shape=jax.ShapeDtypeStruct((total, 1, HD), jnp.float32),
        input_output_aliases={2: 0},
        compiler_params=pltpu.CompilerParams(
            dimension_semantics=("parallel", "arbitrary")),
    )(block_tables, context_lens, fill_out.reshape(total, 1, HD), qbd,
      kcur2.reshape(n_gen, 1, HD), vcur2.reshape(n_gen, 1, HD), kcg, vcg)
    return out.reshape(total, HD)


def kernel(q, k, v, k_cache, v_cache, slot_mapping, cu_seqlens_q,
           context_lens, block_tables):
    total = q.shape[0]
    n_gen = block_tables.shape[0]
    num_fill = total - n_gen
    num_seqs = cu_seqlens_q.shape[0] - 1
    seq_len = num_fill // num_seqs

    q2 = q.reshape(total, HD)
    k2 = k.reshape(total, HD)
    v2 = v.reshape(total, HD)

    fill_out = _fill_attention(q2, k2, v2, seq_len, num_seqs, total)
    out = _gen_attention(fill_out, q2[num_fill:], k2[num_fill:], v2[num_fill:],
                         k_cache.reshape(-1, HD), v_cache.reshape(-1, HD),
                         block_tables, context_lens, num_fill)
    return out


# speedup vs baseline: 1.0713x; 1.0531x over previous
import functools

import jax
import jax.numpy as jnp
from jax.experimental import pallas as pl
from jax.experimental.pallas import tpu as pltpu

NUM_HEADS = 16
HEAD_DIM = 128
HD = NUM_HEADS * HEAD_DIM
SCALE = 0.08838834764831845
LOG2E = 1.4426950408889634
PAGE = 16

BQ = 256
BK = 256
GBK = 512
GSUB = 2
NEG = -1e30


def _fill_kernel(q_ref, k_ref, v_ref, o_ref, kbf_ref, vbf_ref):
    qb = pl.program_id(2)

    @pl.when(qb == 0)
    def _cast():
        kbf_ref[...] = k_ref[...].astype(jnp.bfloat16)
        vbf_ref[...] = v_ref[...].astype(jnp.bfloat16)

    qv = (q_ref[...] * (SCALE * LOG2E)).astype(jnp.bfloat16)
    row = qb * BQ + jax.lax.broadcasted_iota(jnp.int32, (BQ, BK), 0)
    col0 = jax.lax.broadcasted_iota(jnp.int32, (BQ, BK), 1)

    def pair_step(i, carry):
        acc, m, l = carry
        for t in (0, 1):
            j = 2 * i + t
            kj = kbf_ref[pl.ds(j * BK, BK), :]
            s = jax.lax.dot_general(qv, kj, (((1,), (1,)), ((), ())),
                                    preferred_element_type=jnp.float32)
            s = jnp.where(j * BK + col0 <= row, s, NEG)
            m_new = jnp.maximum(m, jnp.max(s, axis=1, keepdims=True))
            alpha = jnp.exp2(m - m_new)
            p = jnp.exp2(s - m_new)
            l = l * alpha + jnp.sum(p, axis=1, keepdims=True)
            vj = vbf_ref[pl.ds(j * BK, BK), :]
            acc = acc * alpha + jax.lax.dot_general(
                p.astype(jnp.bfloat16), vj, (((1,), (0,)), ((), ())),
                preferred_element_type=jnp.float32)
            m = m_new
        return acc, m, l

    acc0 = jnp.zeros((BQ, HEAD_DIM), jnp.float32)
    m0 = jnp.full((BQ, 1), NEG, jnp.float32)
    l0 = jnp.zeros((BQ, 1), jnp.float32)
    acc, m, l = jax.lax.fori_loop(0, qb // 2 + 1, pair_step, (acc0, m0, l0))
    o_ref[...] = acc / l


def _fill_attention(q2, k2, v2, seq_len, num_seqs, total):
    qblocks = seq_len // BQ
    grid = (num_seqs, NUM_HEADS, qblocks)
    return pl.pallas_call(
        _fill_kernel,
        grid=grid,
        in_specs=[
            pl.BlockSpec((BQ, HEAD_DIM), lambda s, h, qb: (s * qblocks + qb, h)),
            pl.BlockSpec((seq_len, HEAD_DIM), lambda s, h, qb: (s, h)),
            pl.BlockSpec((seq_len, HEAD_DIM), lambda s, h, qb: (s, h)),
        ],
        out_specs=pl.BlockSpec((BQ, HEAD_DIM), lambda s, h, qb: (s * qblocks + qb, h)),
        out_shape=jax.ShapeDtypeStruct((total, HD), jnp.float32),
        scratch_shapes=[
            pltpu.VMEM((seq_len, HEAD_DIM), jnp.bfloat16),
            pltpu.VMEM((seq_len, HEAD_DIM), jnp.bfloat16),
        ],
        compiler_params=pltpu.CompilerParams(
            dimension_semantics=("parallel", "arbitrary", "arbitrary")),
    )(q2, k2, v2)


def _gen_kernel(bt_ref, cl_ref, io_ref, qbd_ref, kcur_ref, vcur_ref,
                *rest, n_chunks):
    del io_ref
    kc_refs = rest[:GSUB]
    vc_refs = rest[GSUB:2 * GSUB]
    o_ref, acc_ref, m_ref, l_ref = rest[2 * GSUB:]
    g = pl.program_id(0)
    c = pl.program_id(1)
    ctx = cl_ref[g]
    qbd = qbd_ref[0]

    @pl.when(c == 0)
    def _init():
        logit_cur = jax.lax.dot_general(
            qbd, kcur_ref[0], (((0,), (1,)), ((), ())),
            preferred_element_type=jnp.float32)
        m_ref[...] = jnp.broadcast_to(logit_cur, (NUM_HEADS, HEAD_DIM))
        l_ref[...] = jnp.ones((NUM_HEADS, HEAD_DIM), jnp.float32)
        acc_ref[...] = jnp.broadcast_to(vcur_ref[0], (NUM_HEADS, HD))

    for i, (kc_ref, vc_ref) in enumerate(zip(kc_refs, vc_refs)):
        kblk = kc_ref[0]
        s_hs = jax.lax.dot_general(qbd, kblk, (((0,), (1,)), ((), ())),
                                   preferred_element_type=jnp.float32)
        pos = ((c * GSUB + i) * GBK
               + jax.lax.broadcasted_iota(jnp.int32, (NUM_HEADS, GBK), 1))
        s_hs = jnp.where(pos < ctx - 1, s_hs, NEG)

        m_prev = m_ref[:, :1]
        l_prev = l_ref[:, :1]
        m_new = jnp.maximum(m_prev, jnp.max(s_hs, axis=1, keepdims=True))
        alpha = jnp.exp2(m_prev - m_new)
        p = jnp.exp2(s_hs - m_new)
        l_new = l_prev * alpha + jnp.sum(p, axis=1, keepdims=True)
        vblk = vc_ref[0]
        pv = jax.lax.dot_general(p, vblk, (((1,), (0,)), ((), ())),
                                 preferred_element_type=jnp.float32)
        acc_new = acc_ref[...] * alpha + pv
        acc_ref[...] = acc_new
        m_ref[...] = jnp.broadcast_to(m_new, (NUM_HEADS, HEAD_DIM))
        l_ref[...] = jnp.broadcast_to(l_new, (NUM_HEADS, HEAD_DIM))

    @pl.when(c == n_chunks - 1)
    def _done():
        hsel = jax.lax.broadcasted_iota(jnp.int32, (NUM_HEADS, HD), 1) // HEAD_DIM
        hrow = jax.lax.broadcasted_iota(jnp.int32, (NUM_HEADS, HD), 0)
        picked = jnp.where(hsel == hrow, acc_ref[...] / l_ref[:, :1], 0.0)
        o_ref[0] = jnp.sum(picked, axis=0, keepdims=True)


def _gen_attention(fill_out, qg, kcur2, vcur2, kc, vc, block_tables,
                   context_lens, num_fill):
    n_gen, bpseq = block_tables.shape
    ctx_cap = bpseq * PAGE
    n_chunks = ctx_cap // (GBK * GSUB)
    per = GBK // PAGE
    ngroups = kc.shape[0] // GBK
    kcg = kc.reshape(ngroups, GBK, HD)
    vcg = vc.reshape(ngroups, GBK, HD)

    ridx = jnp.arange(HD, dtype=jnp.int32)
    hmask = (ridx[:, None] // HEAD_DIM) == jnp.arange(NUM_HEADS, dtype=jnp.int32)[None, :]
    qbd = jnp.where(hmask[None], (qg * (SCALE * LOG2E))[:, :, None], 0.0)

    def cache_map_i(i):
        def cache_map(g, c, bt, cl):
            return (bt[g, (c * GSUB + i) * per] // per, 0, 0)
        return cache_map

    cache_specs = [pl.BlockSpec((1, GBK, HD), cache_map_i(i))
                   for i in range(GSUB)]
    grid_spec = pltpu.PrefetchScalarGridSpec(
        num_scalar_prefetch=2,
        grid=(n_gen, n_chunks),
        in_specs=[
            pl.BlockSpec(memory_space=pl.ANY),
            pl.BlockSpec((1, HD, NUM_HEADS), lambda g, c, bt, cl: (g, 0, 0)),
            pl.BlockSpec((1, 1, HD), lambda g, c, bt, cl: (g, 0, 0)),
            pl.BlockSpec((1, 1, HD), lambda g, c, bt, cl: (g, 0, 0)),
        ] + cache_specs + cache_specs,
        out_specs=pl.BlockSpec((1, 1, HD), lambda g, c, bt, cl: (num_fill + g, 0, 0)),
        scratch_shapes=[
            pltpu.VMEM((NUM_HEADS, HD), jnp.float32),
            pltpu.VMEM((NUM_HEADS, HEAD_DIM), jnp.float32),
            pltpu.VMEM((NUM_HEADS, HEAD_DIM), jnp.float32),
        ],
    )
    total = num_fill + n_gen
    out = pl.pallas_call(
        functools.partial(_gen_kernel, n_chunks=n_chunks),
        grid_spec=grid_spec,
        out_shape=jax.ShapeDtypeStruct((total, 1, HD), jnp.float32),
        input_output_aliases={2: 0},
        compiler_params=pltpu.CompilerParams(
            dimension_semantics=("parallel", "arbitrary")),
    )(block_tables, context_lens, fill_out.reshape(total, 1, HD), qbd,
      kcur2.reshape(n_gen, 1, HD), vcur2.reshape(n_gen, 1, HD),
      *([kcg] * GSUB), *([vcg] * GSUB))
    return out.reshape(total, HD)


def kernel(q, k, v, k_cache, v_cache, slot_mapping, cu_seqlens_q,
           context_lens, block_tables):
    total = q.shape[0]
    n_gen = block_tables.shape[0]
    num_fill = total - n_gen
    num_seqs = cu_seqlens_q.shape[0] - 1
    seq_len = num_fill // num_seqs

    q2 = q.reshape(total, HD)
    k2 = k.reshape(total, HD)
    v2 = v.reshape(total, HD)

    fill_out = _fill_attention(q2, k2, v2, seq_len, num_seqs, total)
    out = _gen_attention(fill_out, q2[num_fill:], k2[num_fill:], v2[num_fill:],
                         k_cache.reshape(-1, HD), v_cache.reshape(-1, HD),
                         block_tables, context_lens, num_fill)
    return out


# speedup vs baseline: 1.3930x; 1.3004x over previous
import functools

import jax
import jax.numpy as jnp
from jax.experimental import pallas as pl
from jax.experimental.pallas import tpu as pltpu

NUM_HEADS = 16
HEAD_DIM = 128
HD = NUM_HEADS * HEAD_DIM
SCALE = 0.08838834764831845
LOG2E = 1.4426950408889634
PAGE = 16

BQ = 512
BK = 512
GBK = 512
GSUB = 2
NEG = -1e30


def _fill_kernel(q_ref, k_ref, v_ref, o_ref, kbf_ref, vbf_ref):
    qb = pl.program_id(2)

    @pl.when(qb == 0)
    def _cast():
        kbf_ref[...] = k_ref[...].astype(jnp.bfloat16)
        vbf_ref[...] = v_ref[...].astype(jnp.bfloat16)

    qv = (q_ref[...] * (SCALE * LOG2E)).astype(jnp.bfloat16)
    row = qb * BQ + jax.lax.broadcasted_iota(jnp.int32, (BQ, BK), 0)
    col0 = jax.lax.broadcasted_iota(jnp.int32, (BQ, BK), 1)

    def pair_step(i, carry):
        acc, m, l = carry
        for t in (0, 1):
            j = 2 * i + t
            kj = kbf_ref[pl.ds(j * BK, BK), :]
            s = jax.lax.dot_general(qv, kj, (((1,), (1,)), ((), ())),
                                    preferred_element_type=jnp.float32)
            s = jnp.where(j * BK + col0 <= row, s, NEG)
            m_new = jnp.maximum(m, jnp.max(s, axis=1, keepdims=True))
            alpha = jnp.exp2(m - m_new)
            p = jnp.exp2(s - m_new)
            l = l * alpha + jnp.sum(p, axis=1, keepdims=True)
            vj = vbf_ref[pl.ds(j * BK, BK), :]
            acc = acc * alpha + jax.lax.dot_general(
                p.astype(jnp.bfloat16), vj, (((1,), (0,)), ((), ())),
                preferred_element_type=jnp.float32)
            m = m_new
        return acc, m, l

    acc0 = jnp.zeros((BQ, HEAD_DIM), jnp.float32)
    m0 = jnp.full((BQ, 1), NEG, jnp.float32)
    l0 = jnp.zeros((BQ, 1), jnp.float32)
    acc, m, l = jax.lax.fori_loop(0, qb // 2 + 1, pair_step, (acc0, m0, l0))
    o_ref[...] = acc / l


def _fill_attention(q2, k2, v2, seq_len, num_seqs, total):
    qblocks = seq_len // BQ
    grid = (num_seqs, NUM_HEADS, qblocks)
    return pl.pallas_call(
        _fill_kernel,
        grid=grid,
        in_specs=[
            pl.BlockSpec((BQ, HEAD_DIM), lambda s, h, qb: (s * qblocks + qb, h)),
            pl.BlockSpec((seq_len, HEAD_DIM), lambda s, h, qb: (s, h)),
            pl.BlockSpec((seq_len, HEAD_DIM), lambda s, h, qb: (s, h)),
        ],
        out_specs=pl.BlockSpec((BQ, HEAD_DIM), lambda s, h, qb: (s * qblocks + qb, h)),
        out_shape=jax.ShapeDtypeStruct((total, HD), jnp.float32),
        scratch_shapes=[
            pltpu.VMEM((seq_len, HEAD_DIM), jnp.bfloat16),
            pltpu.VMEM((seq_len, HEAD_DIM), jnp.bfloat16),
        ],
        compiler_params=pltpu.CompilerParams(
            dimension_semantics=("parallel", "arbitrary", "arbitrary")),
    )(q2, k2, v2)


def _gen_kernel(bt_ref, cl_ref, io_ref, qbd_ref, kcur_ref, vcur_ref,
                *rest, n_chunks):
    del io_ref
    kc_refs = rest[:GSUB]
    vc_refs = rest[GSUB:2 * GSUB]
    o_ref, acc_ref, m_ref, l_ref = rest[2 * GSUB:]
    g = pl.program_id(0)
    c = pl.program_id(1)
    ctx = cl_ref[g]
    qbd = qbd_ref[0]

    @pl.when(c == 0)
    def _init():
        logit_cur = jax.lax.dot_general(
            qbd, kcur_ref[0], (((0,), (1,)), ((), ())),
            preferred_element_type=jnp.float32)
        m_ref[...] = jnp.broadcast_to(logit_cur, (NUM_HEADS, HEAD_DIM))
        l_ref[...] = jnp.ones((NUM_HEADS, HEAD_DIM), jnp.float32)
        acc_ref[...] = jnp.broadcast_to(vcur_ref[0], (NUM_HEADS, HD))

    for i, (kc_ref, vc_ref) in enumerate(zip(kc_refs, vc_refs)):
        kblk = kc_ref[0]
        s_hs = jax.lax.dot_general(qbd, kblk, (((0,), (1,)), ((), ())),
                                   preferred_element_type=jnp.float32)
        pos = ((c * GSUB + i) * GBK
               + jax.lax.broadcasted_iota(jnp.int32, (NUM_HEADS, GBK), 1))
        s_hs = jnp.where(pos < ctx - 1, s_hs, NEG)

        m_prev = m_ref[:, :1]
        l_prev = l_ref[:, :1]
        m_new = jnp.maximum(m_prev, jnp.max(s_hs, axis=1, keepdims=True))
        alpha = jnp.exp2(m_prev - m_new)
        p = jnp.exp2(s_hs - m_new)
        l_new = l_prev * alpha + jnp.sum(p, axis=1, keepdims=True)
        vblk = vc_ref[0]
        pv = jax.lax.dot_general(p, vblk, (((1,), (0,)), ((), ())),
                                 preferred_element_type=jnp.float32)
        acc_new = acc_ref[...] * alpha + pv
        acc_ref[...] = acc_new
        m_ref[...] = jnp.broadcast_to(m_new, (NUM_HEADS, HEAD_DIM))
        l_ref[...] = jnp.broadcast_to(l_new, (NUM_HEADS, HEAD_DIM))

    @pl.when(c == n_chunks - 1)
    def _done():
        hsel = jax.lax.broadcasted_iota(jnp.int32, (NUM_HEADS, HD), 1) // HEAD_DIM
        hrow = jax.lax.broadcasted_iota(jnp.int32, (NUM_HEADS, HD), 0)
        picked = jnp.where(hsel == hrow, acc_ref[...] / l_ref[:, :1], 0.0)
        o_ref[0] = jnp.sum(picked, axis=0, keepdims=True)


def _gen_attention(fill_out, qg, kcur2, vcur2, kc, vc, block_tables,
                   context_lens, num_fill):
    n_gen, bpseq = block_tables.shape
    ctx_cap = bpseq * PAGE
    n_chunks = ctx_cap // (GBK * GSUB)
    per = GBK // PAGE
    ngroups = kc.shape[0] // GBK
    kcg = kc.reshape(ngroups, GBK, HD)
    vcg = vc.reshape(ngroups, GBK, HD)

    ridx = jnp.arange(HD, dtype=jnp.int32)
    hmask = (ridx[:, None] // HEAD_DIM) == jnp.arange(NUM_HEADS, dtype=jnp.int32)[None, :]
    qbd = jnp.where(hmask[None], (qg * (SCALE * LOG2E))[:, :, None], 0.0)

    def cache_map_i(i):
        def cache_map(g, c, bt, cl):
            return (bt[g, (c * GSUB + i) * per] // per, 0, 0)
        return cache_map

    cache_specs = [pl.BlockSpec((1, GBK, HD), cache_map_i(i))
                   for i in range(GSUB)]
    grid_spec = pltpu.PrefetchScalarGridSpec(
        num_scalar_prefetch=2,
        grid=(n_gen, n_chunks),
        in_specs=[
            pl.BlockSpec(memory_space=pl.ANY),
            pl.BlockSpec((1, HD, NUM_HEADS), lambda g, c, bt, cl: (g, 0, 0)),
            pl.BlockSpec((1, 1, HD), lambda g, c, bt, cl: (g, 0, 0)),
            pl.BlockSpec((1, 1, HD), lambda g, c, bt, cl: (g, 0, 0)),
        ] + cache_specs + cache_specs,
        out_specs=pl.BlockSpec((1, 1, HD), lambda g, c, bt, cl: (num_fill + g, 0, 0)),
        scratch_shapes=[
            pltpu.VMEM((NUM_HEADS, HD), jnp.float32),
            pltpu.VMEM((NUM_HEADS, HEAD_DIM), jnp.float32),
            pltpu.VMEM((NUM_HEADS, HEAD_DIM), jnp.float32),
        ],
    )
    total = num_fill + n_gen
    out = pl.pallas_call(
        functools.partial(_gen_kernel, n_chunks=n_chunks),
        grid_spec=grid_spec,
        out_shape=jax.ShapeDtypeStruct((total, 1, HD), jnp.float32),
        input_output_aliases={2: 0},
        compiler_params=pltpu.CompilerParams(
            dimension_semantics=("parallel", "arbitrary")),
    )(block_tables, context_lens, fill_out.reshape(total, 1, HD), qbd,
      kcur2.reshape(n_gen, 1, HD), vcur2.reshape(n_gen, 1, HD),
      *([kcg] * GSUB), *([vcg] * GSUB))
    return out.reshape(total, HD)


def kernel(q, k, v, k_cache, v_cache, slot_mapping, cu_seqlens_q,
           context_lens, block_tables):
    total = q.shape[0]
    n_gen = block_tables.shape[0]
    num_fill = total - n_gen
    num_seqs = cu_seqlens_q.shape[0] - 1
    seq_len = num_fill // num_seqs

    q2 = q.reshape(total, HD)
    k2 = k.reshape(total, HD)
    v2 = v.reshape(total, HD)

    fill_out = _fill_attention(q2, k2, v2, seq_len, num_seqs, total)
    out = _gen_attention(fill_out, q2[num_fill:], k2[num_fill:], v2[num_fill:],
                         k_cache.reshape(-1, HD), v_cache.reshape(-1, HD),
                         block_tables, context_lens, num_fill)
    return out
